# baseline (speedup 1.0000x reference)
import functools

import jax
import jax.numpy as jnp
from jax import lax
from jax.experimental import pallas as pl
from jax.experimental.pallas import tpu as pltpu
from jax.experimental.pallas import tpu_sc as plsc

CYCLE = 3
D = 128
BATCH = 16384
HIST = 200
NTOT = BATCH * HIST

K = 4
NCOMBO = CYCLE ** K
CD = K * D

NC = 2
NS = 16
NW = NC * NS
PER_W = NTOT // NW
PER_WC = PER_W // K

CHC = 32
BSUB = 8
BLKC = BSUB * CHC
BLK = BLKC * K
NBLK = PER_W // BLK
NGRP = BLKC // 16
RING = 4


def _body(idx_hbm, table4_hbm, out_hbm, idxb0, idxb1, cidx, rows,
          is0, is1, gs0, gs1, gs2, gs3, os0, os1, os2, os3):
    idxbs = [idxb0, idxb1]
    isem = [is0, is1]
    gsem = [gs0, gs1, gs2, gs3]
    osem = [os0, os1, os2, os3]

    cid = lax.axis_index("c")
    sid = lax.axis_index("s")
    wid = sid * NC + cid

    tab_sh = table4_hbm

    idx_row0 = wid * PER_W
    out_row0 = wid * PER_WC

    def idx_src(g):
        return idx_hbm.at[pl.ds(idx_row0 + g * BLK, BLK)]

    def out_dst(gidx):
        return out_hbm.at[pl.ds(out_row0 + gidx * CHC, CHC)]

    def wait_idx(bb):
        pltpu.make_async_copy(idx_src(0), idxbs[bb], isem[bb]).wait()

    def wait_gat(p):
        pltpu.make_async_copy(
            tab_sh.at[cidx.at[0, pl.ds(0, CHC)]], rows.at[p], gsem[p]
        ).wait()

    def wait_out(p):
        pltpu.make_async_copy(rows.at[p], out_dst(0), osem[p]).wait()

    pltpu.async_copy(idx_src(0), idxbs[0], isem[0])

    def blk2(g2, carry):
        for bb in range(2):
            g = g2 * 2 + bb
            wait_idx(bb)

            def combine(grp, c):
                acc = jnp.mod(idxbs[bb][pl.ds(grp * 64, 16)], CYCLE)
                for t in range(1, K):
                    m = jnp.mod(idxbs[bb][pl.ds(grp * 64 + t * 16, 16)], CYCLE)
                    acc = acc + m * (CYCLE ** t)
                cidx[bb, pl.ds(grp * 16, 16)] = acc
                return c

            lax.fori_loop(0, NGRP, combine, 0)

            for j in range(BSUB):
                p = j % RING
                pm = (j - 1) % RING
                if bb == 0 and j < RING:
                    @pl.when(g2 > 0)
                    def _():
                        wait_out(p)
                else:
                    wait_out(p)
                pltpu.async_copy(
                    tab_sh.at[cidx.at[bb, pl.ds(j * CHC, CHC)]],
                    rows.at[p], gsem[p],
                )
                if j == 0:
                    @pl.when(g > 0)
                    def _():
                        wait_gat(pm)
                        pltpu.async_copy(
                            rows.at[pm], out_dst(g * BSUB - 1), osem[pm]
                        )
                else:
                    wait_gat(pm)
                    pltpu.async_copy(
                        rows.at[pm], out_dst(g * BSUB + j - 1), osem[pm]
                    )
                if j == 0:
                    @pl.when(g < NBLK - 1)
                    def _():
                        pltpu.async_copy(
                            idx_src(g + 1), idxbs[1 - bb], isem[1 - bb]
                        )
        return carry

    lax.fori_loop(0, NBLK // 2, blk2, 0)

    last = NBLK * BSUB - 1
    pl_last = (BSUB - 1) % RING
    wait_gat(pl_last)
    pltpu.async_copy(rows.at[pl_last], out_dst(last), osem[pl_last])
    for p in range(RING):
        wait_out(p)


@jax.jit
def _run(idxf, table):
    c = jnp.arange(NCOMBO)
    table4 = jnp.concatenate(
        [table[(c // (CYCLE ** t)) % CYCLE] for t in range(K)], axis=1
    )
    mesh = plsc.VectorSubcoreMesh(core_axis_name="c", subcore_axis_name="s")
    return pl.kernel(
        _body,
        out_type=jax.ShapeDtypeStruct((NTOT // K, CD), jnp.float32),
        mesh=mesh,
        scratch_types=[
            pltpu.VMEM((BLK,), jnp.int32),
            pltpu.VMEM((BLK,), jnp.int32),
            pltpu.VMEM((2, BLKC), jnp.int32),
            pltpu.VMEM((RING, CHC, CD), jnp.float32),
            pltpu.SemaphoreType.DMA,
            pltpu.SemaphoreType.DMA,
            pltpu.SemaphoreType.DMA,
            pltpu.SemaphoreType.DMA,
            pltpu.SemaphoreType.DMA,
            pltpu.SemaphoreType.DMA,
            pltpu.SemaphoreType.DMA,
            pltpu.SemaphoreType.DMA,
            pltpu.SemaphoreType.DMA,
            pltpu.SemaphoreType.DMA,
        ],
    )(idxf, table4)


def kernel(idx, table):
    idxf = idx.reshape(-1, 16, K).swapaxes(1, 2).reshape(NTOT)
    out = _run(idxf, table)
    return out.reshape(BATCH, HIST, D)


# speedup vs baseline: 5.2403x; 5.2403x over previous
import functools

import jax
import jax.numpy as jnp
from jax import lax
from jax.experimental import pallas as pl
from jax.experimental.pallas import tpu as pltpu
from jax.experimental.pallas import tpu_sc as plsc

CYCLE = 3
D = 128
BATCH = 16384
HIST = 200
NTOT = BATCH * HIST

NC = 2
NS = 16
NW = NC * NS
PER_W = NTOT // NW

CH = 128
BSUB = 8
BLK = BSUB * CH
NBLK = PER_W // BLK
IDX_ROWS_W = PER_W // CH
RING = 4


def _body(idx_hbm, table_hbm, out_hbm, tab_sh, idxb, rows,
          is0, is1, gs0, gs1, gs2, gs3, os0, os1, os2, os3):
    isem = [is0, is1]
    gsem = [gs0, gs1, gs2, gs3]
    osem = [os0, os1, os2, os3]

    cid = lax.axis_index("c")
    sid = lax.axis_index("s")
    wid = sid * NC + cid

    @pl.when(sid == 0)
    def _():
        pltpu.sync_copy(table_hbm, tab_sh)

    plsc.subcore_barrier()

    idx_row0 = wid * PER_W
    out_row0 = wid * PER_W

    def idx_src(g):
        return idx_hbm.at[pl.ds(idx_row0 + g * BLK, BLK)]

    def out_dst(gidx):
        return out_hbm.at[pl.ds(out_row0 + gidx * CH, CH)]

    def wait_idx(bb):
        pltpu.make_async_copy(idx_src(0), idxb.at[bb], isem[bb]).wait()

    def wait_gat(p):
        pltpu.make_async_copy(
            tab_sh.at[idxb.at[0, pl.ds(0, CH)]], rows.at[p], gsem[p]
        ).wait()

    def wait_out(p):
        pltpu.make_async_copy(rows.at[p], out_dst(0), osem[p]).wait()

    pltpu.async_copy(idx_src(0), idxb.at[0], isem[0])

    def blk2(g2, carry):
        for bb in range(2):
            g = g2 * 2 + bb
            wait_idx(bb)

            def wrap(i, c):
                v = idxb[bb, pl.ds(i * 16, 16)]
                idxb[bb, pl.ds(i * 16, 16)] = jnp.mod(v, CYCLE)
                return c

            lax.fori_loop(0, BLK // 16, wrap, 0)

            for j in range(BSUB):
                p = j % RING
                pm = (j - 1) % RING
                if bb == 0 and j < RING:
                    @pl.when(g2 > 0)
                    def _():
                        wait_out(p)
                else:
                    wait_out(p)
                pltpu.async_copy(
                    tab_sh.at[idxb.at[bb, pl.ds(j * CH, CH)]],
                    rows.at[p], gsem[p],
                )
                if j == 0:
                    @pl.when(g > 0)
                    def _():
                        wait_gat(pm)
                        pltpu.async_copy(
                            rows.at[pm], out_dst(g * BSUB - 1), osem[pm]
                        )
                else:
                    wait_gat(pm)
                    pltpu.async_copy(
                        rows.at[pm], out_dst(g * BSUB + j - 1), osem[pm]
                    )
                if j == 0:
                    @pl.when(g < NBLK - 1)
                    def _():
                        pltpu.async_copy(
                            idx_src(g + 1), idxb.at[1 - bb], isem[1 - bb]
                        )
        return carry

    lax.fori_loop(0, NBLK // 2, blk2, 0)

    last = NBLK * BSUB - 1
    pl_last = (BSUB - 1) % RING
    wait_gat(pl_last)
    pltpu.async_copy(rows.at[pl_last], out_dst(last), osem[pl_last])
    for p in range(RING):
        wait_out(p)


@jax.jit
def _run(idx2, table):
    mesh = plsc.VectorSubcoreMesh(core_axis_name="c", subcore_axis_name="s")
    return pl.kernel(
        _body,
        out_type=jax.ShapeDtypeStruct((NTOT, D), jnp.float32),
        mesh=mesh,
        scratch_types=[
            pltpu.VMEM_SHARED((CYCLE, D), jnp.float32),
            pltpu.VMEM((2, BLK), jnp.int32),
            pltpu.VMEM((RING, CH, D), jnp.float32),
            pltpu.SemaphoreType.DMA,
            pltpu.SemaphoreType.DMA,
            pltpu.SemaphoreType.DMA,
            pltpu.SemaphoreType.DMA,
            pltpu.SemaphoreType.DMA,
            pltpu.SemaphoreType.DMA,
            pltpu.SemaphoreType.DMA,
            pltpu.SemaphoreType.DMA,
            pltpu.SemaphoreType.DMA,
            pltpu.SemaphoreType.DMA,
        ],
    )(idx2, table)


def kernel(idx, table):
    out = _run(idx.reshape(NTOT), table)
    return out.reshape(BATCH, HIST, D)


# speedup vs baseline: 6.0247x; 1.1497x over previous
import functools

import jax
import jax.numpy as jnp
from jax import lax
from jax.experimental import pallas as pl
from jax.experimental.pallas import tpu as pltpu
from jax.experimental.pallas import tpu_sc as plsc

CYCLE = 3
D = 128
BATCH = 16384
HIST = 200
NTOT = BATCH * HIST

NC = 2
NS = 16
NW = NC * NS
PER_W = NTOT // NW

CH = 128
BSUB = 8
BLK = BSUB * CH
NBLK = PER_W // BLK
IDX_ROWS_W = PER_W // CH
RING = 4


def _body(idx_hbm, table_hbm, out_hbm, tab_sh, idxb, rows,
          is0, is1, gs0, gs1, gs2, gs3, os0, os1, os2, os3):
    isem = [is0, is1]
    gsem = [gs0, gs1, gs2, gs3]
    osem = [os0, os1, os2, os3]

    cid = lax.axis_index("c")
    sid = lax.axis_index("s")
    wid = sid * NC + cid

    @pl.when(sid == 0)
    def _():
        pltpu.sync_copy(table_hbm, tab_sh)

    plsc.subcore_barrier()

    idx_row0 = wid * PER_W
    out_row0 = wid * PER_W

    def idx_src(g):
        return idx_hbm.at[pl.ds(idx_row0 + g * BLK, BLK)]

    def out_dst(gidx):
        return out_hbm.at[pl.ds(out_row0 + gidx * CH, CH)]

    def wait_idx(bb):
        pltpu.make_async_copy(idx_src(0), idxb.at[bb], isem[bb]).wait()

    def wait_gat(p):
        pltpu.make_async_copy(
            tab_sh.at[idxb.at[0, pl.ds(0, CH)]], rows.at[p], gsem[p]
        ).wait()

    def wait_out(p):
        pltpu.make_async_copy(rows.at[p], out_dst(0), osem[p]).wait()

    pltpu.async_copy(idx_src(0), idxb.at[0], isem[0])

    def blk2(g2, carry):
        for bb in range(2):
            g = g2 * 2 + bb
            wait_idx(bb)

            def wrap(i, c):
                v = idxb[bb, pl.ds(i * 16, 16)]
                idxb[bb, pl.ds(i * 16, 16)] = jnp.mod(v, CYCLE)
                return c

            lax.fori_loop(0, BLK // 16, wrap, 0)

            for j in range(BSUB):
                p = j % RING
                pm = (j - 1) % RING
                if bb == 0 and j < RING:
                    @pl.when(g2 > 0)
                    def _():
                        wait_out(p)
                else:
                    wait_out(p)
                pltpu.async_copy(
                    tab_sh.at[idxb.at[bb, pl.ds(j * CH, CH)]],
                    rows.at[p], gsem[p],
                )
                pm2 = (j - 2) % RING
                if j < 2:
                    @pl.when(g > 0)
                    def _(pm2=pm2):
                        wait_gat(pm2)
                        pltpu.async_copy(
                            rows.at[pm2], out_dst(g * BSUB + j - 2), osem[pm2]
                        )
                else:
                    wait_gat(pm2)
                    pltpu.async_copy(
                        rows.at[pm2], out_dst(g * BSUB + j - 2), osem[pm2]
                    )
                if j == 1:
                    @pl.when(g < NBLK - 1)
                    def _():
                        pltpu.async_copy(
                            idx_src(g + 1), idxb.at[1 - bb], isem[1 - bb]
                        )
        return carry

    lax.fori_loop(0, NBLK // 2, blk2, 0)

    last = NBLK * BSUB - 1
    for back in (1, 0):
        pq = (last - back) % RING
        wait_gat(pq)
        pltpu.async_copy(rows.at[pq], out_dst(last - back), osem[pq])
    for p in range(RING):
        wait_out(p)


@jax.jit
def _run(idx2, table):
    mesh = plsc.VectorSubcoreMesh(core_axis_name="c", subcore_axis_name="s")
    return pl.kernel(
        _body,
        out_type=jax.ShapeDtypeStruct((NTOT, D), jnp.float32),
        mesh=mesh,
        scratch_types=[
            pltpu.VMEM_SHARED((CYCLE, D), jnp.float32),
            pltpu.VMEM((2, BLK), jnp.int32),
            pltpu.VMEM((RING, CH, D), jnp.float32),
            pltpu.SemaphoreType.DMA,
            pltpu.SemaphoreType.DMA,
            pltpu.SemaphoreType.DMA,
            pltpu.SemaphoreType.DMA,
            pltpu.SemaphoreType.DMA,
            pltpu.SemaphoreType.DMA,
            pltpu.SemaphoreType.DMA,
            pltpu.SemaphoreType.DMA,
            pltpu.SemaphoreType.DMA,
            pltpu.SemaphoreType.DMA,
        ],
    )(idx2, table)


def kernel(idx, table):
    out = _run(idx.reshape(NTOT), table)
    return out.reshape(BATCH, HIST, D)
